# Initial kernel scaffold; baseline (speedup 1.0000x reference)
import jax
import jax.numpy as jnp
from jax.experimental import pallas as pl


def kernel(x, token_emb, pos_emb):
    raise NotImplementedError("write your pallas kernel here")


# speedup vs baseline: 4.1131x; 4.1131x over previous
import functools

import jax
import jax.numpy as jnp
from jax import lax
from jax.experimental import pallas as pl
from jax.experimental.pallas import tpu as pltpu
from jax.experimental.pallas import tpu_sc as plsc

_NUM_WORKERS = 32
_LANES = 16


def kernel(x, token_emb, pos_emb):
    B, S = x.shape
    V, D = token_emb.shape
    HALF = S // 2
    SEQ_PER_W = B // _NUM_WORKERS

    x2 = x.reshape(B * 2, HALF).astype(jnp.int32)

    mesh = plsc.VectorSubcoreMesh(core_axis_name="c", subcore_axis_name="s")

    @functools.partial(
        pl.kernel,
        mesh=mesh,
        out_type=jax.ShapeDtypeStruct((B * S, D), jnp.float32),
        scratch_types=[
            pltpu.VMEM((2 * SEQ_PER_W, HALF), jnp.int32),
            pltpu.VMEM((S, D), jnp.float32),
            pltpu.VMEM((S, D), jnp.float32),
        ],
    )
    def run(tok_hbm, idx_hbm, pos_hbm, out_hbm, idx_v, pos_v, buf):
        wid = lax.axis_index("s") * 2 + lax.axis_index("c")
        seq_base = wid * SEQ_PER_W
        pltpu.sync_copy(idx_hbm.at[pl.ds(seq_base * 2, 2 * SEQ_PER_W)], idx_v)
        pltpu.sync_copy(pos_hbm, pos_v)

        @pl.loop(0, SEQ_PER_W)
        def _(q):
            pltpu.sync_copy(tok_hbm.at[idx_v.at[2 * q]], buf.at[pl.ds(0, HALF)])
            pltpu.sync_copy(tok_hbm.at[idx_v.at[2 * q + 1]],
                            buf.at[pl.ds(HALF, HALF)])

            @pl.loop(0, S)
            def _(r):
                for c in range(D // _LANES):
                    sl = pl.ds(c * _LANES, _LANES)
                    plsc.addupdate(buf.at[r, sl], pos_v[r, sl])

            pltpu.sync_copy(buf, out_hbm.at[pl.ds((seq_base + q) * S, S)])

    out = run(token_emb, x2, pos_emb)
    return out.reshape(B, S, D)


# speedup vs baseline: 7.5960x; 1.8468x over previous
import functools

import jax
import jax.numpy as jnp
from jax import lax
from jax.experimental import pallas as pl
from jax.experimental.pallas import tpu as pltpu
from jax.experimental.pallas import tpu_sc as plsc

_NUM_WORKERS = 32
_LANES = 16


def kernel(x, token_emb, pos_emb):
    B, S = x.shape
    V, D = token_emb.shape
    HALF = S // 2
    SEQ_PER_W = B // _NUM_WORKERS

    x2 = x.reshape(B * 2, HALF).astype(jnp.int32)

    mesh = plsc.VectorSubcoreMesh(core_axis_name="c", subcore_axis_name="s")

    @functools.partial(
        pl.kernel,
        mesh=mesh,
        out_type=jax.ShapeDtypeStruct((B * S, D), jnp.float32),
        scratch_types=[
            pltpu.VMEM((2 * SEQ_PER_W, HALF), jnp.int32),
            pltpu.VMEM((S, D), jnp.float32),
            pltpu.VMEM((S, D), jnp.float32),
            pltpu.VMEM((S, D), jnp.float32),
            pltpu.SemaphoreType.DMA,
            pltpu.SemaphoreType.DMA,
            pltpu.SemaphoreType.DMA,
            pltpu.SemaphoreType.DMA,
        ],
    )
    def run(tok_hbm, idx_hbm, pos_hbm, out_hbm, idx_v, pos_v,
            buf0, buf1, gsem0, gsem1, wsem0, wsem1):
        bufs = (buf0, buf1)
        gsems = (gsem0, gsem1)
        wsems = (wsem0, wsem1)
        wid = lax.axis_index("s") * 2 + lax.axis_index("c")
        seq_base = wid * SEQ_PER_W
        pltpu.sync_copy(idx_hbm.at[pl.ds(seq_base * 2, 2 * SEQ_PER_W)], idx_v)
        pltpu.sync_copy(pos_hbm, pos_v)

        def issue_gather(blk, b):
            pltpu.async_copy(tok_hbm.at[idx_v.at[2 * blk]],
                             bufs[b].at[pl.ds(0, HALF)], gsems[b])
            pltpu.async_copy(tok_hbm.at[idx_v.at[2 * blk + 1]],
                             bufs[b].at[pl.ds(HALF, HALF)], gsems[b])

        def wait_gather(blk, b):
            pltpu.make_async_copy(tok_hbm.at[idx_v.at[2 * blk]],
                                  bufs[b].at[pl.ds(0, HALF)], gsems[b]).wait()
            pltpu.make_async_copy(tok_hbm.at[idx_v.at[2 * blk + 1]],
                                  bufs[b].at[pl.ds(HALF, HALF)], gsems[b]).wait()

        def issue_writeback(blk, b):
            pltpu.async_copy(bufs[b], out_hbm.at[pl.ds((seq_base + blk) * S, S)],
                             wsems[b])

        def wait_writeback(b):
            pltpu.make_async_copy(bufs[b], out_hbm.at[pl.ds(0, S)],
                                  wsems[b]).wait()

        def add_pos(b):
            buf = bufs[b]

            @pl.loop(0, S)
            def _(r):
                for c in range(D // _LANES):
                    sl = pl.ds(c * _LANES, _LANES)
                    plsc.addupdate(buf.at[r, sl], pos_v[r, sl])

        issue_gather(0, 0)

        @pl.loop(0, SEQ_PER_W // 2)
        def _(t):
            for b in range(2):
                blk = 2 * t + b
                if b == 0:
                    @pl.when(t > 0)
                    def _():
                        wait_writeback(1)
                        issue_gather(blk + 1, 1)

                    @pl.when(t == 0)
                    def _():
                        issue_gather(blk + 1, 1)
                else:
                    @pl.when(blk + 1 < SEQ_PER_W)
                    def _():
                        wait_writeback(0)
                        issue_gather(blk + 1, 0)
                wait_gather(blk, b)
                add_pos(b)
                issue_writeback(blk, b)

        wait_writeback(0)
        wait_writeback(1)

    out = run(token_emb, x2, pos_emb)
    return out.reshape(B, S, D)
